# Initial kernel scaffold; baseline (speedup 1.0000x reference)
import jax
import jax.numpy as jnp
from jax.experimental import pallas as pl


def kernel(h, e, pos_enc, edge_index, params):
    raise NotImplementedError("write your pallas kernel here")


# speedup vs baseline: 25.9960x; 25.9960x over previous
import functools

import jax
import jax.numpy as jnp
from jax import lax
from jax.experimental import pallas as pl
from jax.experimental.pallas import tpu as pltpu
from jax.experimental.pallas import tpu_sc as plsc

_f32 = jnp.float32

_NC = 2
_NS = 16
_NW = _NC * _NS
_CHK = 80

_H = 8
_DK = 16


def _ln(x, g, b):
    m = jnp.mean(x, axis=-1, keepdims=True)
    xc = x - m
    v = jnp.mean(xc * xc, axis=-1, keepdims=True)
    return xc * lax.rsqrt(v + 1e-5) * g + b


def _head_sum_mat(d):
    r = lax.broadcasted_iota(jnp.int32, (d, _H), 0)
    c = lax.broadcasted_iota(jnp.int32, (d, _H), 1)
    return (r // _DK == c).astype(_f32)


def _head_expand_mat(rows, d):
    r = lax.broadcasted_iota(jnp.int32, (rows, d), 0)
    c = lax.broadcasted_iota(jnp.int32, (rows, d), 1)
    return (c // _DK == r).astype(_f32)


def _dot(a, b):
    return jnp.dot(a, b, preferred_element_type=_f32)


def _pre_body(x0_ref, w0_ref, b0_ref, wq_ref, bq_ref, wk_ref, bk_ref,
              wv_ref, bv_ref, h_ref, tkv_ref, tq_ref):
    x = x0_ref[...]
    h = _dot(x, w0_ref[...]) + b0_ref[...]
    h_ref[...] = h
    k = _dot(h, wk_ref[...]) + bk_ref[...]
    v = _dot(h, wv_ref[...]) + bv_ref[...]
    tkv_ref[...] = jnp.concatenate([k, v], axis=1)
    tq_ref[...] = _dot(h, wq_ref[...]) + bq_ref[...]


def _pre_call(x0, w0, b0, qkvw):
    n, d = x0.shape[0], w0.shape[1]
    tn = 1000
    grid = (n // tn,)
    row = lambda i: (i, 0)
    full = lambda a: pl.BlockSpec(a.shape, lambda i: (0,) * a.ndim)
    return pl.pallas_call(
        _pre_body,
        grid=grid,
        in_specs=[pl.BlockSpec((tn, x0.shape[1]), row), full(w0), full(b0)]
        + [full(a) for a in qkvw],
        out_specs=[pl.BlockSpec((tn, d), row), pl.BlockSpec((tn, 2 * d), row),
                   pl.BlockSpec((tn, d), row)],
        out_shape=[jax.ShapeDtypeStruct((n, d), _f32),
                   jax.ShapeDtypeStruct((n, 2 * d), _f32),
                   jax.ShapeDtypeStruct((n, d), _f32)],
    )(x0, w0, b0, *qkvw)


def _edge_body(uniform_e, e_ref, gkv_ref, gq_ref, we_ref, be_ref, woe_ref,
               boe_ref, wf1_ref, bf1_ref, wf2_ref, bf2_ref, g1_ref, b1_ref,
               g2_ref, b2_ref, e3_ref, wvw_ref):
    d = gq_ref.shape[1]
    k = gkv_ref[:, :d]
    v = gkv_ref[:, d:]
    q = gq_ref[...]
    if uniform_e:
        e_row = e_ref[...]
        ep_row = _dot(e_row, we_ref[...]) + be_ref[...]
        e = jnp.broadcast_to(e_row, k.shape)
        ep = jnp.broadcast_to(ep_row, k.shape)
    else:
        e = e_ref[...]
        ep = _dot(e, we_ref[...]) + be_ref[...]
    score = k * q * ep * 0.25
    logits = jnp.clip(_dot(score, _head_sum_mat(d)), -5.0, 5.0)
    w = jnp.exp(logits)
    wfull = _dot(w, _head_expand_mat(_H, d))
    wvw_ref[...] = jnp.concatenate([wfull * v, wfull], axis=1)
    e_o = _dot(score, woe_ref[...]) + boe_ref[...]
    e2 = _ln(e + e_o, g1_ref[...], b1_ref[...])
    f = jnp.maximum(_dot(e2, wf1_ref[...]) + bf1_ref[...], 0.0)
    ef = _dot(f, wf2_ref[...]) + bf2_ref[...]
    e3_ref[...] = _ln(e2 + ef, g2_ref[...], b2_ref[...])


def _edge_call(e_arg, gkv, gq, ew, uniform_e):
    eedges, d = gq.shape
    t = 1280
    grid = (eedges // t,)
    row = lambda i: (i, 0)
    zero = lambda i: (0, 0)
    full = lambda a: pl.BlockSpec(a.shape, lambda i: (0,) * a.ndim)
    e_spec = (pl.BlockSpec((1, d), zero) if uniform_e
              else pl.BlockSpec((t, d), row))
    return pl.pallas_call(
        functools.partial(_edge_body, uniform_e),
        grid=grid,
        in_specs=[e_spec, pl.BlockSpec((t, 2 * d), row),
                  pl.BlockSpec((t, d), row)] + [full(a) for a in ew],
        out_specs=[pl.BlockSpec((t, d), row),
                   pl.BlockSpec((t, 2 * d), row)],
        out_shape=[jax.ShapeDtypeStruct((eedges, d), _f32),
                   jax.ShapeDtypeStruct((eedges, 2 * d), _f32)],
    )(e_arg, gkv, gq, *ew)


def _node_core(h, accn_ref, accd_ref, woh_ref, boh_ref, wf1_ref, bf1_ref,
               wf2_ref, bf2_ref, g1_ref, b1_ref, g2_ref, b2_ref):
    num = accn_ref[0] + accn_ref[1]
    denf = accd_ref[0] + accd_ref[1] + 1e-6
    h_att = num / denf
    h_o = _dot(h_att, woh_ref[...]) + boh_ref[...]
    h2 = _ln(h + h_o, g1_ref[...], b1_ref[...])
    f = jnp.maximum(_dot(h2, wf1_ref[...]) + bf1_ref[...], 0.0)
    hf = _dot(f, wf2_ref[...]) + bf2_ref[...]
    return _ln(h2 + hf, g2_ref[...], b2_ref[...])


def _node_body(h_ref, accn_ref, accd_ref, woh_ref, boh_ref, wf1_ref, bf1_ref,
               wf2_ref, bf2_ref, g1_ref, b1_ref, g2_ref, b2_ref, wq_ref,
               bq_ref, wk_ref, bk_ref, wv_ref, bv_ref, h3_ref, tkv_ref,
               tq_ref):
    h3 = _node_core(h_ref[...], accn_ref, accd_ref, woh_ref, boh_ref, wf1_ref,
                    bf1_ref, wf2_ref, bf2_ref, g1_ref, b1_ref, g2_ref, b2_ref)
    h3_ref[...] = h3
    k = _dot(h3, wk_ref[...]) + bk_ref[...]
    v = _dot(h3, wv_ref[...]) + bv_ref[...]
    tkv_ref[...] = jnp.concatenate([k, v], axis=1)
    tq_ref[...] = _dot(h3, wq_ref[...]) + bq_ref[...]


def _node_call(h, accn2, accd2, nw, qkvw):
    n, d = h.shape
    tn = 1000
    grid = (n // tn,)
    row = lambda i: (i, 0)
    full = lambda a: pl.BlockSpec(a.shape, lambda i: (0,) * a.ndim)
    return pl.pallas_call(
        _node_body,
        grid=grid,
        in_specs=[pl.BlockSpec((tn, d), row),
                  pl.BlockSpec((2, tn, d), lambda i: (0, i, 0)),
                  pl.BlockSpec((2, tn, d), lambda i: (0, i, 0))]
        + [full(a) for a in nw] + [full(a) for a in qkvw],
        out_specs=[pl.BlockSpec((tn, d), row), pl.BlockSpec((tn, 2 * d), row),
                   pl.BlockSpec((tn, d), row)],
        out_shape=[jax.ShapeDtypeStruct((n, d), _f32),
                   jax.ShapeDtypeStruct((n, 2 * d), _f32),
                   jax.ShapeDtypeStruct((n, d), _f32)],
    )(h, accn2, accd2, *nw, *qkvw)


def _node_final_body(h_ref, accn_ref, accd_ref, woh_ref, boh_ref, wf1_ref,
                     bf1_ref, wf2_ref, bf2_ref, g1_ref, b1_ref, g2_ref,
                     b2_ref, m1_ref, c1_ref, m2_ref, c2_ref, m3_ref, c3_ref,
                     nout_ref, hsum_ref):
    h3 = _node_core(h_ref[...], accn_ref, accd_ref, woh_ref, boh_ref, wf1_ref,
                    bf1_ref, wf2_ref, bf2_ref, g1_ref, b1_ref, g2_ref, b2_ref)
    x = jnp.maximum(_dot(h3, m1_ref[...]) + c1_ref[...], 0.0)
    x = jnp.maximum(_dot(x, m2_ref[...]) + c2_ref[...], 0.0)
    nout_ref[...] = _dot(x, m3_ref[...]) + c3_ref[...]

    @pl.when(pl.program_id(0) == 0)
    def _():
        hsum_ref[...] = jnp.zeros_like(hsum_ref)

    hsum_ref[...] += jnp.sum(h3, axis=0, keepdims=True)


def _node_final_call(h, accn2, accd2, nw, mlpw):
    n, d = h.shape
    tn = 1000
    grid = (n // tn,)
    row = lambda i: (i, 0)
    full = lambda a: pl.BlockSpec(a.shape, lambda i: (0,) * a.ndim)
    return pl.pallas_call(
        _node_final_body,
        grid=grid,
        in_specs=[pl.BlockSpec((tn, d), row),
                  pl.BlockSpec((2, tn, d), lambda i: (0, i, 0)),
                  pl.BlockSpec((2, tn, d), lambda i: (0, i, 0))]
        + [full(a) for a in nw] + [full(a) for a in mlpw],
        out_specs=[pl.BlockSpec((tn, 3), row),
                   pl.BlockSpec((1, d), lambda i: (0, 0))],
        out_shape=[jax.ShapeDtypeStruct((n, 3), _f32),
                   jax.ShapeDtypeStruct((1, d), _f32)],
    )(h, accn2, accd2, *nw, *mlpw)


def _graph_body(n_nodes, hsum_ref, m1_ref, c1_ref, m2_ref, c2_ref, m3_ref,
                c3_ref, gout_ref):
    hg = hsum_ref[...] * (1.0 / n_nodes)
    x = jnp.maximum(_dot(hg, m1_ref[...]) + c1_ref[...], 0.0)
    x = jnp.maximum(_dot(x, m2_ref[...]) + c2_ref[...], 0.0)
    gout_ref[...] = _dot(x, m3_ref[...]) + c3_ref[...]


def _graph_call(hsum, mlpw, n_nodes):
    return pl.pallas_call(
        functools.partial(_graph_body, n_nodes),
        out_shape=jax.ShapeDtypeStruct((1, 3), _f32),
    )(hsum, *mlpw)


def _sc_gather(tkv, tq, src, dst):
    e = src.shape[0]
    n, dkv = tkv.shape
    d = tq.shape[1]
    epw = e // _NW
    chunks = epw // _CHK
    mesh = plsc.VectorSubcoreMesh(core_axis_name="c", subcore_axis_name="s",
                                  num_cores=_NC, num_subcores=_NS)

    @functools.partial(
        pl.kernel,
        out_type=[jax.ShapeDtypeStruct((e, dkv), _f32),
                  jax.ShapeDtypeStruct((e, d), _f32)],
        mesh=mesh,
        scratch_types=[pltpu.VMEM((_CHK,), jnp.int32),
                       pltpu.VMEM((_CHK,), jnp.int32),
                       pltpu.VMEM((_CHK, dkv), _f32),
                       pltpu.VMEM((_CHK, d), _f32),
                       pltpu.SemaphoreType.DMA,
                       pltpu.SemaphoreType.DMA],
    )
    def k(tkv_hbm, tq_hbm, src_hbm, dst_hbm, gkv_hbm, gq_hbm, idx_s, idx_d,
          bkv, bq, sem1, sem2):
        wid = lax.axis_index("s") * _NC + lax.axis_index("c")
        base = wid * epw

        def body(j, carry):
            off = base + j * _CHK
            pltpu.sync_copy(src_hbm.at[pl.ds(off, _CHK)], idx_s)
            pltpu.sync_copy(dst_hbm.at[pl.ds(off, _CHK)], idx_d)
            cp1 = pltpu.async_copy(tkv_hbm.at[idx_s], bkv, sem1)
            cp2 = pltpu.async_copy(tq_hbm.at[idx_d], bq, sem2)
            cp1.wait()
            cp2.wait()
            pltpu.sync_copy(bkv, gkv_hbm.at[pl.ds(off, _CHK)])
            pltpu.sync_copy(bq, gq_hbm.at[pl.ds(off, _CHK)])
            return carry

        lax.fori_loop(0, chunks, body, 0)

    return k(tkv, tq, src, dst)


def _sc_scatter(wvw, coff, dst, n, zrow, seq):
    e = wvw.shape[0]
    d = 128
    epw = e // _NW
    chunks = epw // _CHK
    rows = n // _NS
    mesh = plsc.VectorSubcoreMesh(core_axis_name="c", subcore_axis_name="s",
                                  num_cores=_NC, num_subcores=_NS)

    @functools.partial(
        pl.kernel,
        out_type=jax.ShapeDtypeStruct((_NC * n, d), _f32),
        mesh=mesh,
        scratch_types=[pltpu.VMEM_SHARED((n, d), _f32),
                       pltpu.VMEM((_CHK,), jnp.int32),
                       pltpu.VMEM((_CHK,), jnp.int32),
                       pltpu.VMEM((_CHK, d), _f32)],
    )
    def k(wvw_hbm, dst_hbm, zrow_hbm, seq_hbm, acc_hbm, acc, idx, ridx, buf):
        cid = lax.axis_index("c")
        sid = lax.axis_index("s")
        zchunks = rows // _CHK

        pltpu.sync_copy(zrow_hbm, buf)

        def zbody(t, carry):
            roff = sid * rows + t * _CHK
            pltpu.sync_copy(seq_hbm.at[pl.ds(roff, _CHK)], ridx)
            pltpu.sync_copy(buf, acc.at[ridx])
            return carry

        lax.fori_loop(0, zchunks, zbody, 0)
        plsc.subcore_barrier()

        wid = sid * _NC + cid
        base = wid * epw

        def body(j, carry):
            off = base + j * _CHK
            pltpu.sync_copy(dst_hbm.at[pl.ds(off, _CHK)], idx)
            pltpu.sync_copy(wvw_hbm.at[pl.ds(off, _CHK), pl.ds(coff, d)], buf)
            pltpu.sync_copy(buf, acc.at[idx], add=True)
            return carry

        lax.fori_loop(0, chunks, body, 0)
        plsc.subcore_barrier()

        def obody(t, carry):
            roff = sid * rows + t * _CHK
            pltpu.sync_copy(seq_hbm.at[pl.ds(roff, _CHK)], ridx)
            pltpu.sync_copy(acc.at[ridx], buf)
            pltpu.sync_copy(buf, acc_hbm.at[pl.ds(cid * n + roff, _CHK)])
            return carry

        lax.fori_loop(0, zchunks, obody, 0)

    return k(wvw, dst, zrow, seq)


def kernel(h, e, pos_enc, edge_index, params):
    n, d = h.shape[0], params['Wh'].shape[1]
    del e
    src = edge_index[0]
    dst = edge_index[1]
    lw = params['layers']
    n_layers = lw['Wq'].shape[0]
    r2 = lambda a: a.reshape(1, -1)

    pad = d // 8 - h.shape[1] - pos_enc.shape[1]
    x0 = jnp.concatenate([h, pos_enc, jnp.zeros((n, pad), _f32)], axis=1)
    w0 = jnp.concatenate(
        [params['Wh'], params['Wpe'], jnp.zeros((pad, d), _f32)], axis=0)
    b0 = r2(params['bh'] + params['bpe'])

    qkvw = lambda l: (lw['Wq'][l], r2(lw['bq'][l]), lw['Wk'][l],
                      r2(lw['bk'][l]), lw['Wv'][l], r2(lw['bv'][l]))
    hcur, tkv, tq = _pre_call(x0, w0, b0, qkvw(0))

    e_cur = params['Wee'][0:1] + r2(params['bee'])
    npad = ((n + _CHK * _NS - 1) // (_CHK * _NS)) * (_CHK * _NS)
    zrow = jnp.zeros((_CHK, d), _f32)
    seq = jnp.arange(npad, dtype=jnp.int32)

    nout = gout = None
    for l in range(n_layers):
        gkv, gq = _sc_gather(tkv, tq, src, dst)
        ew = (lw['We'][l], r2(lw['be'][l]), lw['Woe'][l], r2(lw['boe'][l]),
              lw['Wf1e'][l], r2(lw['bf1e'][l]), lw['Wf2e'][l],
              r2(lw['bf2e'][l]), r2(lw['ln1eg'][l]), r2(lw['ln1eb'][l]),
              r2(lw['ln2eg'][l]), r2(lw['ln2eb'][l]))
        e_cur, wvw = _edge_call(e_cur, gkv, gq, ew, uniform_e=(l == 0))
        accn2 = _sc_scatter(wvw, 0, dst, npad, zrow, seq).reshape(
            _NC, npad, -1)
        accd2 = _sc_scatter(wvw, d, dst, npad, zrow, seq).reshape(
            _NC, npad, -1)
        nw = (lw['Woh'][l], r2(lw['boh'][l]), lw['Wf1h'][l],
              r2(lw['bf1h'][l]), lw['Wf2h'][l], r2(lw['bf2h'][l]),
              r2(lw['ln1hg'][l]), r2(lw['ln1hb'][l]), r2(lw['ln2hg'][l]),
              r2(lw['ln2hb'][l]))
        if l < n_layers - 1:
            hcur, tkv, tq = _node_call(hcur, accn2, accd2, nw, qkvw(l + 1))
        else:
            mlpn = tuple(x for wb in params['mlp_n']
                         for x in (wb[0], r2(wb[1])))
            nout, hsum = _node_final_call(hcur, accn2, accd2, nw, mlpn)
            mlpg = tuple(x for wb in params['mlp_g']
                         for x in (wb[0], r2(wb[1])))
            gout = _graph_call(hsum, mlpg, n)

    return nout, gout.reshape(gout.shape[-1])


# speedup vs baseline: 30.8074x; 1.1851x over previous
import functools

import jax
import jax.numpy as jnp
from jax import lax
from jax.experimental import pallas as pl
from jax.experimental.pallas import tpu as pltpu
from jax.experimental.pallas import tpu_sc as plsc

_f32 = jnp.float32

_NC = 2
_NS = 16
_NW = _NC * _NS
_CHK = 80

_H = 8
_DK = 16


def _ln(x, g, b):
    m = jnp.mean(x, axis=-1, keepdims=True)
    xc = x - m
    v = jnp.mean(xc * xc, axis=-1, keepdims=True)
    return xc * lax.rsqrt(v + 1e-5) * g + b


def _head_sum_mat(d):
    r = lax.broadcasted_iota(jnp.int32, (d, _H), 0)
    c = lax.broadcasted_iota(jnp.int32, (d, _H), 1)
    return (r // _DK == c).astype(_f32)


def _head_expand_mat(rows, d):
    r = lax.broadcasted_iota(jnp.int32, (rows, d), 0)
    c = lax.broadcasted_iota(jnp.int32, (rows, d), 1)
    return (c // _DK == r).astype(_f32)


def _dot(a, b):
    return jnp.dot(a, b, preferred_element_type=_f32)


def _pre_body(x0_ref, w0_ref, b0_ref, wq_ref, bq_ref, wk_ref, bk_ref,
              wv_ref, bv_ref, h_ref, tkv_ref, tq_ref):
    x = x0_ref[...]
    h = _dot(x, w0_ref[...]) + b0_ref[...]
    h_ref[...] = h
    k = _dot(h, wk_ref[...]) + bk_ref[...]
    v = _dot(h, wv_ref[...]) + bv_ref[...]
    tkv_ref[...] = jnp.concatenate([k, v], axis=1)
    tq_ref[...] = _dot(h, wq_ref[...]) + bq_ref[...]


def _pre_call(x0, w0, b0, qkvw):
    n, d = x0.shape[0], w0.shape[1]
    tn = 1000
    grid = (n // tn,)
    row = lambda i: (i, 0)
    full = lambda a: pl.BlockSpec(a.shape, lambda i: (0,) * a.ndim)
    return pl.pallas_call(
        _pre_body,
        grid=grid,
        in_specs=[pl.BlockSpec((tn, x0.shape[1]), row), full(w0), full(b0)]
        + [full(a) for a in qkvw],
        out_specs=[pl.BlockSpec((tn, d), row), pl.BlockSpec((tn, 2 * d), row),
                   pl.BlockSpec((tn, d), row)],
        out_shape=[jax.ShapeDtypeStruct((n, d), _f32),
                   jax.ShapeDtypeStruct((n, 2 * d), _f32),
                   jax.ShapeDtypeStruct((n, d), _f32)],
    )(x0, w0, b0, *qkvw)


def _edge_body(uniform_e, e_ref, gkv_ref, gq_ref, we_ref, be_ref, woe_ref,
               boe_ref, wf1_ref, bf1_ref, wf2_ref, bf2_ref, g1_ref, b1_ref,
               g2_ref, b2_ref, e3_ref, wvw_ref):
    d = gq_ref.shape[1]
    k = gkv_ref[:, :d]
    v = gkv_ref[:, d:]
    q = gq_ref[...]
    if uniform_e:
        e_row = e_ref[...]
        ep_row = _dot(e_row, we_ref[...]) + be_ref[...]
        e = jnp.broadcast_to(e_row, k.shape)
        ep = jnp.broadcast_to(ep_row, k.shape)
    else:
        e = e_ref[...]
        ep = _dot(e, we_ref[...]) + be_ref[...]
    score = k * q * ep * 0.25
    logits = jnp.clip(_dot(score, _head_sum_mat(d)), -5.0, 5.0)
    w = jnp.exp(logits)
    wfull = _dot(w, _head_expand_mat(_H, d))
    wvw_ref[...] = jnp.concatenate([wfull * v, wfull], axis=1)
    e_o = _dot(score, woe_ref[...]) + boe_ref[...]
    e2 = _ln(e + e_o, g1_ref[...], b1_ref[...])
    f = jnp.maximum(_dot(e2, wf1_ref[...]) + bf1_ref[...], 0.0)
    ef = _dot(f, wf2_ref[...]) + bf2_ref[...]
    e3_ref[...] = _ln(e2 + ef, g2_ref[...], b2_ref[...])


def _edge_call(e_arg, gkv, gq, ew, uniform_e):
    eedges, d = gq.shape
    t = 1280
    grid = (eedges // t,)
    row = lambda i: (i, 0)
    zero = lambda i: (0, 0)
    full = lambda a: pl.BlockSpec(a.shape, lambda i: (0,) * a.ndim)
    e_spec = (pl.BlockSpec((1, d), zero) if uniform_e
              else pl.BlockSpec((t, d), row))
    return pl.pallas_call(
        functools.partial(_edge_body, uniform_e),
        grid=grid,
        in_specs=[e_spec, pl.BlockSpec((t, 2 * d), row),
                  pl.BlockSpec((t, d), row)] + [full(a) for a in ew],
        out_specs=[pl.BlockSpec((t, d), row),
                   pl.BlockSpec((t, 2 * d), row)],
        out_shape=[jax.ShapeDtypeStruct((eedges, d), _f32),
                   jax.ShapeDtypeStruct((eedges, 2 * d), _f32)],
    )(e_arg, gkv, gq, *ew)


def _node_core(h, accn_ref, accd_ref, woh_ref, boh_ref, wf1_ref, bf1_ref,
               wf2_ref, bf2_ref, g1_ref, b1_ref, g2_ref, b2_ref):
    num = accn_ref[0] + accn_ref[1]
    denf = accd_ref[0] + accd_ref[1] + 1e-6
    h_att = num / denf
    h_o = _dot(h_att, woh_ref[...]) + boh_ref[...]
    h2 = _ln(h + h_o, g1_ref[...], b1_ref[...])
    f = jnp.maximum(_dot(h2, wf1_ref[...]) + bf1_ref[...], 0.0)
    hf = _dot(f, wf2_ref[...]) + bf2_ref[...]
    return _ln(h2 + hf, g2_ref[...], b2_ref[...])


def _node_body(h_ref, accn_ref, accd_ref, woh_ref, boh_ref, wf1_ref, bf1_ref,
               wf2_ref, bf2_ref, g1_ref, b1_ref, g2_ref, b2_ref, wq_ref,
               bq_ref, wk_ref, bk_ref, wv_ref, bv_ref, h3_ref, tkv_ref,
               tq_ref):
    h3 = _node_core(h_ref[...], accn_ref, accd_ref, woh_ref, boh_ref, wf1_ref,
                    bf1_ref, wf2_ref, bf2_ref, g1_ref, b1_ref, g2_ref, b2_ref)
    h3_ref[...] = h3
    k = _dot(h3, wk_ref[...]) + bk_ref[...]
    v = _dot(h3, wv_ref[...]) + bv_ref[...]
    tkv_ref[...] = jnp.concatenate([k, v], axis=1)
    tq_ref[...] = _dot(h3, wq_ref[...]) + bq_ref[...]


def _node_call(h, accn2, accd2, nw, qkvw):
    n, d = h.shape
    tn = 1000
    grid = (n // tn,)
    row = lambda i: (i, 0)
    full = lambda a: pl.BlockSpec(a.shape, lambda i: (0,) * a.ndim)
    return pl.pallas_call(
        _node_body,
        grid=grid,
        in_specs=[pl.BlockSpec((tn, d), row),
                  pl.BlockSpec((2, tn, d), lambda i: (0, i, 0)),
                  pl.BlockSpec((2, tn, d), lambda i: (0, i, 0))]
        + [full(a) for a in nw] + [full(a) for a in qkvw],
        out_specs=[pl.BlockSpec((tn, d), row), pl.BlockSpec((tn, 2 * d), row),
                   pl.BlockSpec((tn, d), row)],
        out_shape=[jax.ShapeDtypeStruct((n, d), _f32),
                   jax.ShapeDtypeStruct((n, 2 * d), _f32),
                   jax.ShapeDtypeStruct((n, d), _f32)],
    )(h, accn2, accd2, *nw, *qkvw)


def _node_final_body(h_ref, accn_ref, accd_ref, woh_ref, boh_ref, wf1_ref,
                     bf1_ref, wf2_ref, bf2_ref, g1_ref, b1_ref, g2_ref,
                     b2_ref, m1_ref, c1_ref, m2_ref, c2_ref, m3_ref, c3_ref,
                     nout_ref, hsum_ref):
    h3 = _node_core(h_ref[...], accn_ref, accd_ref, woh_ref, boh_ref, wf1_ref,
                    bf1_ref, wf2_ref, bf2_ref, g1_ref, b1_ref, g2_ref, b2_ref)
    x = jnp.maximum(_dot(h3, m1_ref[...]) + c1_ref[...], 0.0)
    x = jnp.maximum(_dot(x, m2_ref[...]) + c2_ref[...], 0.0)
    nout_ref[...] = _dot(x, m3_ref[...]) + c3_ref[...]

    @pl.when(pl.program_id(0) == 0)
    def _():
        hsum_ref[...] = jnp.zeros_like(hsum_ref)

    hsum_ref[...] += jnp.sum(h3, axis=0, keepdims=True)


def _node_final_call(h, accn2, accd2, nw, mlpw):
    n, d = h.shape
    tn = 1000
    grid = (n // tn,)
    row = lambda i: (i, 0)
    full = lambda a: pl.BlockSpec(a.shape, lambda i: (0,) * a.ndim)
    return pl.pallas_call(
        _node_final_body,
        grid=grid,
        in_specs=[pl.BlockSpec((tn, d), row),
                  pl.BlockSpec((2, tn, d), lambda i: (0, i, 0)),
                  pl.BlockSpec((2, tn, d), lambda i: (0, i, 0))]
        + [full(a) for a in nw] + [full(a) for a in mlpw],
        out_specs=[pl.BlockSpec((tn, 3), row),
                   pl.BlockSpec((1, d), lambda i: (0, 0))],
        out_shape=[jax.ShapeDtypeStruct((n, 3), _f32),
                   jax.ShapeDtypeStruct((1, d), _f32)],
    )(h, accn2, accd2, *nw, *mlpw)


def _graph_body(n_nodes, hsum_ref, m1_ref, c1_ref, m2_ref, c2_ref, m3_ref,
                c3_ref, gout_ref):
    hg = hsum_ref[...] * (1.0 / n_nodes)
    x = jnp.maximum(_dot(hg, m1_ref[...]) + c1_ref[...], 0.0)
    x = jnp.maximum(_dot(x, m2_ref[...]) + c2_ref[...], 0.0)
    gout_ref[...] = _dot(x, m3_ref[...]) + c3_ref[...]


def _graph_call(hsum, mlpw, n_nodes):
    return pl.pallas_call(
        functools.partial(_graph_body, n_nodes),
        out_shape=jax.ShapeDtypeStruct((1, 3), _f32),
    )(hsum, *mlpw)


def _sc_gather(tkv, tq, src, dst):
    e = src.shape[0]
    n, dkv = tkv.shape
    d = tq.shape[1]
    epw = e // _NW
    chunks = epw // _CHK
    mesh = plsc.VectorSubcoreMesh(core_axis_name="c", subcore_axis_name="s",
                                  num_cores=_NC, num_subcores=_NS)

    @functools.partial(
        pl.kernel,
        out_type=[jax.ShapeDtypeStruct((e, dkv), _f32),
                  jax.ShapeDtypeStruct((e, d), _f32)],
        mesh=mesh,
        scratch_types=[pltpu.VMEM((_CHK,), jnp.int32),
                       pltpu.VMEM((_CHK,), jnp.int32),
                       pltpu.VMEM((_CHK, dkv), _f32),
                       pltpu.VMEM((_CHK, d), _f32),
                       pltpu.SemaphoreType.DMA,
                       pltpu.SemaphoreType.DMA],
    )
    def k(tkv_hbm, tq_hbm, src_hbm, dst_hbm, gkv_hbm, gq_hbm, idx_s, idx_d,
          bkv, bq, sem1, sem2):
        wid = lax.axis_index("s") * _NC + lax.axis_index("c")
        base = wid * epw

        def body(j, carry):
            off = base + j * _CHK
            pltpu.sync_copy(src_hbm.at[pl.ds(off, _CHK)], idx_s)
            pltpu.sync_copy(dst_hbm.at[pl.ds(off, _CHK)], idx_d)
            cp1 = pltpu.async_copy(tkv_hbm.at[idx_s], bkv, sem1)
            cp2 = pltpu.async_copy(tq_hbm.at[idx_d], bq, sem2)
            cp1.wait()
            cp2.wait()
            pltpu.sync_copy(bkv, gkv_hbm.at[pl.ds(off, _CHK)])
            pltpu.sync_copy(bq, gq_hbm.at[pl.ds(off, _CHK)])
            return carry

        lax.fori_loop(0, chunks, body, 0)

    return k(tkv, tq, src, dst)


def _sc_scatter(wvw, coff, dst, n, zrow, seq):
    e = wvw.shape[0]
    d = 128
    epw = e // _NW
    chunks = epw // _CHK
    rows = n // _NS
    mesh = plsc.VectorSubcoreMesh(core_axis_name="c", subcore_axis_name="s",
                                  num_cores=_NC, num_subcores=_NS)

    @functools.partial(
        pl.kernel,
        out_type=jax.ShapeDtypeStruct((_NC * n, d), _f32),
        mesh=mesh,
        scratch_types=[pltpu.VMEM_SHARED((n, d), _f32),
                       pltpu.VMEM((_CHK,), jnp.int32),
                       pltpu.VMEM((_CHK,), jnp.int32),
                       pltpu.VMEM((_CHK, d), _f32),
                       pltpu.VMEM((2, _CHK, d), _f32),
                       pltpu.SemaphoreType.DMA((2,))],
    )
    def k(wvw_hbm, dst_hbm, zrow_hbm, seq_hbm, acc_hbm, acc, idx, ridx, buf,
          dbuf, sem):
        cid = lax.axis_index("c")
        sid = lax.axis_index("s")
        zchunks = rows // _CHK

        pltpu.sync_copy(zrow_hbm, buf)

        def zbody(t, carry):
            roff = sid * rows + t * _CHK
            pltpu.sync_copy(seq_hbm.at[pl.ds(roff, _CHK)], ridx)
            pltpu.sync_copy(buf, acc.at[ridx])
            return carry

        lax.fori_loop(0, zchunks, zbody, 0)
        plsc.subcore_barrier()

        wid = sid * _NC + cid
        base = wid * epw

        def _load(j, slot):
            src = wvw_hbm.at[pl.ds(base + j * _CHK, _CHK), pl.ds(coff, d)]
            return pltpu.make_async_copy(src, dbuf.at[slot], sem.at[slot])

        _load(0, 0).start()

        def body(j, carry):
            m = j % 2
            nx = (j + 1) % 2

            @pl.when(j + 1 < chunks)
            def _():
                _load(j + 1, nx).start()

            pltpu.sync_copy(dst_hbm.at[pl.ds(base + j * _CHK, _CHK)], idx)
            _load(j, m).wait()
            pltpu.sync_copy(dbuf.at[m], acc.at[idx], add=True)
            return carry

        lax.fori_loop(0, chunks, body, 0)
        plsc.subcore_barrier()

        def obody(t, carry):
            roff = sid * rows + t * _CHK
            pltpu.sync_copy(seq_hbm.at[pl.ds(roff, _CHK)], ridx)
            pltpu.sync_copy(acc.at[ridx], buf)
            pltpu.sync_copy(buf, acc_hbm.at[pl.ds(cid * n + roff, _CHK)])
            return carry

        lax.fori_loop(0, zchunks, obody, 0)

    return k(wvw, dst, zrow, seq)


def kernel(h, e, pos_enc, edge_index, params):
    n, d = h.shape[0], params['Wh'].shape[1]
    del e
    src = edge_index[0]
    dst = edge_index[1]
    lw = params['layers']
    n_layers = lw['Wq'].shape[0]
    r2 = lambda a: a.reshape(1, -1)

    pad = d // 8 - h.shape[1] - pos_enc.shape[1]
    x0 = jnp.concatenate([h, pos_enc, jnp.zeros((n, pad), _f32)], axis=1)
    w0 = jnp.concatenate(
        [params['Wh'], params['Wpe'], jnp.zeros((pad, d), _f32)], axis=0)
    b0 = r2(params['bh'] + params['bpe'])

    qkvw = lambda l: (lw['Wq'][l], r2(lw['bq'][l]), lw['Wk'][l],
                      r2(lw['bk'][l]), lw['Wv'][l], r2(lw['bv'][l]))
    hcur, tkv, tq = _pre_call(x0, w0, b0, qkvw(0))

    e_cur = params['Wee'][0:1] + r2(params['bee'])
    npad = ((n + _CHK * _NS - 1) // (_CHK * _NS)) * (_CHK * _NS)
    zrow = jnp.zeros((_CHK, d), _f32)
    seq = jnp.arange(npad, dtype=jnp.int32)

    nout = gout = None
    for l in range(n_layers):
        gkv, gq = _sc_gather(tkv, tq, src, dst)
        ew = (lw['We'][l], r2(lw['be'][l]), lw['Woe'][l], r2(lw['boe'][l]),
              lw['Wf1e'][l], r2(lw['bf1e'][l]), lw['Wf2e'][l],
              r2(lw['bf2e'][l]), r2(lw['ln1eg'][l]), r2(lw['ln1eb'][l]),
              r2(lw['ln2eg'][l]), r2(lw['ln2eb'][l]))
        e_cur, wvw = _edge_call(e_cur, gkv, gq, ew, uniform_e=(l == 0))
        accn2 = _sc_scatter(wvw, 0, dst, npad, zrow, seq).reshape(
            _NC, npad, -1)
        accd2 = _sc_scatter(wvw, d, dst, npad, zrow, seq).reshape(
            _NC, npad, -1)
        nw = (lw['Woh'][l], r2(lw['boh'][l]), lw['Wf1h'][l],
              r2(lw['bf1h'][l]), lw['Wf2h'][l], r2(lw['bf2h'][l]),
              r2(lw['ln1hg'][l]), r2(lw['ln1hb'][l]), r2(lw['ln2hg'][l]),
              r2(lw['ln2hb'][l]))
        if l < n_layers - 1:
            hcur, tkv, tq = _node_call(hcur, accn2, accd2, nw, qkvw(l + 1))
        else:
            mlpn = tuple(x for wb in params['mlp_n']
                         for x in (wb[0], r2(wb[1])))
            nout, hsum = _node_final_call(hcur, accn2, accd2, nw, mlpn)
            mlpg = tuple(x for wb in params['mlp_g']
                         for x in (wb[0], r2(wb[1])))
            gout = _graph_call(hsum, mlpg, n)

    return nout, gout.reshape(gout.shape[-1])


# speedup vs baseline: 30.8378x; 1.0010x over previous
import functools

import jax
import jax.numpy as jnp
from jax import lax
from jax.experimental import pallas as pl
from jax.experimental.pallas import tpu as pltpu
from jax.experimental.pallas import tpu_sc as plsc

_f32 = jnp.float32

_NC = 2
_NS = 16
_NW = _NC * _NS
_CHK = 80

_H = 8
_DK = 16


def _ln(x, g, b):
    m = jnp.mean(x, axis=-1, keepdims=True)
    xc = x - m
    v = jnp.mean(xc * xc, axis=-1, keepdims=True)
    return xc * lax.rsqrt(v + 1e-5) * g + b


def _head_sum_mat(d):
    r = lax.broadcasted_iota(jnp.int32, (d, _H), 0)
    c = lax.broadcasted_iota(jnp.int32, (d, _H), 1)
    return (r // _DK == c).astype(_f32)


def _head_expand_mat(rows, d):
    r = lax.broadcasted_iota(jnp.int32, (rows, d), 0)
    c = lax.broadcasted_iota(jnp.int32, (rows, d), 1)
    return (c // _DK == r).astype(_f32)


def _dot(a, b):
    return jnp.dot(a, b, preferred_element_type=_f32)


def _pre_body(x0_ref, w0_ref, b0_ref, wq_ref, bq_ref, wk_ref, bk_ref,
              wv_ref, bv_ref, h_ref, tkv_ref, tq_ref):
    x = x0_ref[...]
    h = _dot(x, w0_ref[...]) + b0_ref[...]
    h_ref[...] = h
    k = _dot(h, wk_ref[...]) + bk_ref[...]
    v = _dot(h, wv_ref[...]) + bv_ref[...]
    tkv_ref[...] = jnp.concatenate([k, v], axis=1)
    tq_ref[...] = _dot(h, wq_ref[...]) + bq_ref[...]


def _pre_call(x0, w0, b0, qkvw):
    n, d = x0.shape[0], w0.shape[1]
    tn = 1000
    grid = (n // tn,)
    row = lambda i: (i, 0)
    full = lambda a: pl.BlockSpec(a.shape, lambda i: (0,) * a.ndim)
    return pl.pallas_call(
        _pre_body,
        grid=grid,
        in_specs=[pl.BlockSpec((tn, x0.shape[1]), row), full(w0), full(b0)]
        + [full(a) for a in qkvw],
        out_specs=[pl.BlockSpec((tn, d), row), pl.BlockSpec((tn, 2 * d), row),
                   pl.BlockSpec((tn, d), row)],
        out_shape=[jax.ShapeDtypeStruct((n, d), _f32),
                   jax.ShapeDtypeStruct((n, 2 * d), _f32),
                   jax.ShapeDtypeStruct((n, d), _f32)],
    )(x0, w0, b0, *qkvw)


def _edge_body(uniform_e, e_ref, gkv_ref, gq_ref, we_ref, be_ref, woe_ref,
               boe_ref, wf1_ref, bf1_ref, wf2_ref, bf2_ref, g1_ref, b1_ref,
               g2_ref, b2_ref, e3_ref, wvw_ref):
    d = gq_ref.shape[1]
    k = gkv_ref[:, :d]
    v = gkv_ref[:, d:]
    q = gq_ref[...]
    if uniform_e:
        e_row = e_ref[...]
        ep_row = _dot(e_row, we_ref[...]) + be_ref[...]
        e = jnp.broadcast_to(e_row, k.shape)
        ep = jnp.broadcast_to(ep_row, k.shape)
    else:
        e = e_ref[...]
        ep = _dot(e, we_ref[...]) + be_ref[...]
    score = k * q * ep * 0.25
    logits = jnp.clip(_dot(score, _head_sum_mat(d)), -5.0, 5.0)
    w = jnp.exp(logits)
    wfull = _dot(w, _head_expand_mat(_H, d))
    wvw_ref[...] = jnp.concatenate([wfull * v, wfull], axis=1)
    e_o = _dot(score, woe_ref[...]) + boe_ref[...]
    e2 = _ln(e + e_o, g1_ref[...], b1_ref[...])
    f = jnp.maximum(_dot(e2, wf1_ref[...]) + bf1_ref[...], 0.0)
    ef = _dot(f, wf2_ref[...]) + bf2_ref[...]
    e3_ref[...] = _ln(e2 + ef, g2_ref[...], b2_ref[...])


def _edge_call(e_arg, gkv, gq, ew, uniform_e):
    eedges, d = gq.shape
    t = 1280
    grid = (eedges // t,)
    row = lambda i: (i, 0)
    zero = lambda i: (0, 0)
    full = lambda a: pl.BlockSpec(a.shape, lambda i: (0,) * a.ndim)
    e_spec = (pl.BlockSpec((1, d), zero) if uniform_e
              else pl.BlockSpec((t, d), row))
    return pl.pallas_call(
        functools.partial(_edge_body, uniform_e),
        grid=grid,
        in_specs=[e_spec, pl.BlockSpec((t, 2 * d), row),
                  pl.BlockSpec((t, d), row)] + [full(a) for a in ew],
        out_specs=[pl.BlockSpec((t, d), row),
                   pl.BlockSpec((t, 2 * d), row)],
        out_shape=[jax.ShapeDtypeStruct((eedges, d), _f32),
                   jax.ShapeDtypeStruct((eedges, 2 * d), _f32)],
    )(e_arg, gkv, gq, *ew)


def _node_core(h, accn_ref, accd_ref, woh_ref, boh_ref, wf1_ref, bf1_ref,
               wf2_ref, bf2_ref, g1_ref, b1_ref, g2_ref, b2_ref):
    num = accn_ref[0] + accn_ref[1]
    denf = accd_ref[0] + accd_ref[1] + 1e-6
    h_att = num / denf
    h_o = _dot(h_att, woh_ref[...]) + boh_ref[...]
    h2 = _ln(h + h_o, g1_ref[...], b1_ref[...])
    f = jnp.maximum(_dot(h2, wf1_ref[...]) + bf1_ref[...], 0.0)
    hf = _dot(f, wf2_ref[...]) + bf2_ref[...]
    return _ln(h2 + hf, g2_ref[...], b2_ref[...])


def _node_body(h_ref, accn_ref, accd_ref, woh_ref, boh_ref, wf1_ref, bf1_ref,
               wf2_ref, bf2_ref, g1_ref, b1_ref, g2_ref, b2_ref, wq_ref,
               bq_ref, wk_ref, bk_ref, wv_ref, bv_ref, h3_ref, tkv_ref,
               tq_ref):
    h3 = _node_core(h_ref[...], accn_ref, accd_ref, woh_ref, boh_ref, wf1_ref,
                    bf1_ref, wf2_ref, bf2_ref, g1_ref, b1_ref, g2_ref, b2_ref)
    h3_ref[...] = h3
    k = _dot(h3, wk_ref[...]) + bk_ref[...]
    v = _dot(h3, wv_ref[...]) + bv_ref[...]
    tkv_ref[...] = jnp.concatenate([k, v], axis=1)
    tq_ref[...] = _dot(h3, wq_ref[...]) + bq_ref[...]


def _node_call(h, accn2, accd2, nw, qkvw):
    n, d = h.shape
    tn = 1000
    grid = (n // tn,)
    row = lambda i: (i, 0)
    full = lambda a: pl.BlockSpec(a.shape, lambda i: (0,) * a.ndim)
    return pl.pallas_call(
        _node_body,
        grid=grid,
        in_specs=[pl.BlockSpec((tn, d), row),
                  pl.BlockSpec((2, tn, d), lambda i: (0, i, 0)),
                  pl.BlockSpec((2, tn, d), lambda i: (0, i, 0))]
        + [full(a) for a in nw] + [full(a) for a in qkvw],
        out_specs=[pl.BlockSpec((tn, d), row), pl.BlockSpec((tn, 2 * d), row),
                   pl.BlockSpec((tn, d), row)],
        out_shape=[jax.ShapeDtypeStruct((n, d), _f32),
                   jax.ShapeDtypeStruct((n, 2 * d), _f32),
                   jax.ShapeDtypeStruct((n, d), _f32)],
    )(h, accn2, accd2, *nw, *qkvw)


def _node_final_body(h_ref, accn_ref, accd_ref, woh_ref, boh_ref, wf1_ref,
                     bf1_ref, wf2_ref, bf2_ref, g1_ref, b1_ref, g2_ref,
                     b2_ref, m1_ref, c1_ref, m2_ref, c2_ref, m3_ref, c3_ref,
                     nout_ref, hsum_ref):
    h3 = _node_core(h_ref[...], accn_ref, accd_ref, woh_ref, boh_ref, wf1_ref,
                    bf1_ref, wf2_ref, bf2_ref, g1_ref, b1_ref, g2_ref, b2_ref)
    x = jnp.maximum(_dot(h3, m1_ref[...]) + c1_ref[...], 0.0)
    x = jnp.maximum(_dot(x, m2_ref[...]) + c2_ref[...], 0.0)
    nout_ref[...] = _dot(x, m3_ref[...]) + c3_ref[...]

    @pl.when(pl.program_id(0) == 0)
    def _():
        hsum_ref[...] = jnp.zeros_like(hsum_ref)

    hsum_ref[...] += jnp.sum(h3, axis=0, keepdims=True)


def _node_final_call(h, accn2, accd2, nw, mlpw):
    n, d = h.shape
    tn = 1000
    grid = (n // tn,)
    row = lambda i: (i, 0)
    full = lambda a: pl.BlockSpec(a.shape, lambda i: (0,) * a.ndim)
    return pl.pallas_call(
        _node_final_body,
        grid=grid,
        in_specs=[pl.BlockSpec((tn, d), row),
                  pl.BlockSpec((2, tn, d), lambda i: (0, i, 0)),
                  pl.BlockSpec((2, tn, d), lambda i: (0, i, 0))]
        + [full(a) for a in nw] + [full(a) for a in mlpw],
        out_specs=[pl.BlockSpec((tn, 3), row),
                   pl.BlockSpec((1, d), lambda i: (0, 0))],
        out_shape=[jax.ShapeDtypeStruct((n, 3), _f32),
                   jax.ShapeDtypeStruct((1, d), _f32)],
    )(h, accn2, accd2, *nw, *mlpw)


def _graph_body(n_nodes, hsum_ref, m1_ref, c1_ref, m2_ref, c2_ref, m3_ref,
                c3_ref, gout_ref):
    hg = hsum_ref[...] * (1.0 / n_nodes)
    x = jnp.maximum(_dot(hg, m1_ref[...]) + c1_ref[...], 0.0)
    x = jnp.maximum(_dot(x, m2_ref[...]) + c2_ref[...], 0.0)
    gout_ref[...] = _dot(x, m3_ref[...]) + c3_ref[...]


def _graph_call(hsum, mlpw, n_nodes):
    return pl.pallas_call(
        functools.partial(_graph_body, n_nodes),
        out_shape=jax.ShapeDtypeStruct((1, 3), _f32),
    )(hsum, *mlpw)


def _sc_gather(tkv, tq, src, dst):
    e = src.shape[0]
    n, dkv = tkv.shape
    d = tq.shape[1]
    epw = e // _NW
    chunks = epw // _CHK
    mesh = plsc.VectorSubcoreMesh(core_axis_name="c", subcore_axis_name="s",
                                  num_cores=_NC, num_subcores=_NS)

    @functools.partial(
        pl.kernel,
        out_type=[jax.ShapeDtypeStruct((e, dkv), _f32),
                  jax.ShapeDtypeStruct((e, d), _f32)],
        mesh=mesh,
        scratch_types=[pltpu.VMEM((_CHK,), jnp.int32),
                       pltpu.VMEM((_CHK,), jnp.int32),
                       pltpu.VMEM((_CHK, dkv), _f32),
                       pltpu.VMEM((_CHK, d), _f32),
                       pltpu.SemaphoreType.DMA,
                       pltpu.SemaphoreType.DMA],
    )
    def k(tkv_hbm, tq_hbm, src_hbm, dst_hbm, gkv_hbm, gq_hbm, idx_s, idx_d,
          bkv, bq, sem1, sem2):
        wid = lax.axis_index("s") * _NC + lax.axis_index("c")
        base = wid * epw

        def body(j, carry):
            off = base + j * _CHK
            pltpu.sync_copy(src_hbm.at[pl.ds(off, _CHK)], idx_s)
            pltpu.sync_copy(dst_hbm.at[pl.ds(off, _CHK)], idx_d)
            cp1 = pltpu.async_copy(tkv_hbm.at[idx_s], bkv, sem1)
            cp2 = pltpu.async_copy(tq_hbm.at[idx_d], bq, sem2)
            cp1.wait()
            cp2.wait()
            pltpu.sync_copy(bkv, gkv_hbm.at[pl.ds(off, _CHK)])
            pltpu.sync_copy(bq, gq_hbm.at[pl.ds(off, _CHK)])
            return carry

        lax.fori_loop(0, chunks, body, 0)

    return k(tkv, tq, src, dst)


def _sc_scatter(wvw, coff, dst, n, zrow, seq):
    e = wvw.shape[0]
    d = 128
    epw = e // _NW
    chunks = epw // _CHK
    rows = n // _NS
    mesh = plsc.VectorSubcoreMesh(core_axis_name="c", subcore_axis_name="s",
                                  num_cores=_NC, num_subcores=_NS)

    @functools.partial(
        pl.kernel,
        out_type=jax.ShapeDtypeStruct((_NC * n, d), _f32),
        mesh=mesh,
        scratch_types=[pltpu.VMEM_SHARED((n, d), _f32),
                       pltpu.VMEM((_CHK,), jnp.int32),
                       pltpu.VMEM((_CHK,), jnp.int32),
                       pltpu.VMEM((_CHK, d), _f32),
                       pltpu.VMEM((2, _CHK, d), _f32),
                       pltpu.VMEM((2, _CHK), jnp.int32),
                       pltpu.SemaphoreType.DMA((2,)),
                       pltpu.SemaphoreType.DMA((2,))],
    )
    def k(wvw_hbm, dst_hbm, zrow_hbm, seq_hbm, acc_hbm, acc, idx, ridx, buf,
          dbuf, idx2, sem, asem):
        cid = lax.axis_index("c")
        sid = lax.axis_index("s")
        zchunks = rows // _CHK

        pltpu.sync_copy(zrow_hbm, buf)

        def zbody(t, carry):
            roff = sid * rows + t * _CHK
            pltpu.sync_copy(seq_hbm.at[pl.ds(roff, _CHK)], ridx)
            pltpu.sync_copy(buf, acc.at[ridx])
            return carry

        lax.fori_loop(0, zchunks, zbody, 0)
        plsc.subcore_barrier()

        wid = sid * _NC + cid
        base = wid * epw

        def _load(j, slot):
            src = wvw_hbm.at[pl.ds(base + j * _CHK, _CHK), pl.ds(coff, d)]
            return pltpu.make_async_copy(src, dbuf.at[slot], sem.at[slot])

        def _add(slot):
            return pltpu.make_async_copy(dbuf.at[slot], acc.at[idx2.at[slot]],
                                         asem.at[slot])

        _load(0, 0).start()

        def body(j, carry):
            m = j % 2
            nx = (j + 1) % 2

            @pl.when(j >= 1)
            def _():
                _add(nx).wait()

            @pl.when(j + 1 < chunks)
            def _():
                _load(j + 1, nx).start()

            pltpu.sync_copy(dst_hbm.at[pl.ds(base + j * _CHK, _CHK)],
                            idx2.at[m])
            _load(j, m).wait()
            _add(m).start(add=True)
            return carry

        lax.fori_loop(0, chunks, body, 0)
        _add((chunks - 1) % 2).wait()
        plsc.subcore_barrier()

        def obody(t, carry):
            roff = sid * rows + t * _CHK
            pltpu.sync_copy(seq_hbm.at[pl.ds(roff, _CHK)], ridx)
            pltpu.sync_copy(acc.at[ridx], buf)
            pltpu.sync_copy(buf, acc_hbm.at[pl.ds(cid * n + roff, _CHK)])
            return carry

        lax.fori_loop(0, zchunks, obody, 0)

    return k(wvw, dst, zrow, seq)


def kernel(h, e, pos_enc, edge_index, params):
    n, d = h.shape[0], params['Wh'].shape[1]
    del e
    src = edge_index[0]
    dst = edge_index[1]
    lw = params['layers']
    n_layers = lw['Wq'].shape[0]
    r2 = lambda a: a.reshape(1, -1)

    pad = d // 8 - h.shape[1] - pos_enc.shape[1]
    x0 = jnp.concatenate([h, pos_enc, jnp.zeros((n, pad), _f32)], axis=1)
    w0 = jnp.concatenate(
        [params['Wh'], params['Wpe'], jnp.zeros((pad, d), _f32)], axis=0)
    b0 = r2(params['bh'] + params['bpe'])

    qkvw = lambda l: (lw['Wq'][l], r2(lw['bq'][l]), lw['Wk'][l],
                      r2(lw['bk'][l]), lw['Wv'][l], r2(lw['bv'][l]))
    hcur, tkv, tq = _pre_call(x0, w0, b0, qkvw(0))

    e_cur = params['Wee'][0:1] + r2(params['bee'])
    npad = ((n + _CHK * _NS - 1) // (_CHK * _NS)) * (_CHK * _NS)
    zrow = jnp.zeros((_CHK, d), _f32)
    seq = jnp.arange(npad, dtype=jnp.int32)

    nout = gout = None
    for l in range(n_layers):
        gkv, gq = _sc_gather(tkv, tq, src, dst)
        ew = (lw['We'][l], r2(lw['be'][l]), lw['Woe'][l], r2(lw['boe'][l]),
              lw['Wf1e'][l], r2(lw['bf1e'][l]), lw['Wf2e'][l],
              r2(lw['bf2e'][l]), r2(lw['ln1eg'][l]), r2(lw['ln1eb'][l]),
              r2(lw['ln2eg'][l]), r2(lw['ln2eb'][l]))
        e_cur, wvw = _edge_call(e_cur, gkv, gq, ew, uniform_e=(l == 0))
        accn2 = _sc_scatter(wvw, 0, dst, npad, zrow, seq).reshape(
            _NC, npad, -1)
        accd2 = _sc_scatter(wvw, d, dst, npad, zrow, seq).reshape(
            _NC, npad, -1)
        nw = (lw['Woh'][l], r2(lw['boh'][l]), lw['Wf1h'][l],
              r2(lw['bf1h'][l]), lw['Wf2h'][l], r2(lw['bf2h'][l]),
              r2(lw['ln1hg'][l]), r2(lw['ln1hb'][l]), r2(lw['ln2hg'][l]),
              r2(lw['ln2hb'][l]))
        if l < n_layers - 1:
            hcur, tkv, tq = _node_call(hcur, accn2, accd2, nw, qkvw(l + 1))
        else:
            mlpn = tuple(x for wb in params['mlp_n']
                         for x in (wb[0], r2(wb[1])))
            nout, hsum = _node_final_call(hcur, accn2, accd2, nw, mlpn)
            mlpg = tuple(x for wb in params['mlp_g']
                         for x in (wb[0], r2(wb[1])))
            gout = _graph_call(hsum, mlpg, n)

    return nout, gout.reshape(gout.shape[-1])


# speedup vs baseline: 31.4962x; 1.0214x over previous
import functools

import jax
import jax.numpy as jnp
from jax import lax
from jax.experimental import pallas as pl
from jax.experimental.pallas import tpu as pltpu
from jax.experimental.pallas import tpu_sc as plsc

_f32 = jnp.float32

_NC = 2
_NS = 16
_NW = _NC * _NS
_CHK = 80

_H = 8
_DK = 16


def _ln(x, g, b):
    m = jnp.mean(x, axis=-1, keepdims=True)
    xc = x - m
    v = jnp.mean(xc * xc, axis=-1, keepdims=True)
    return xc * lax.rsqrt(v + 1e-5) * g + b


def _head_sum_mat(d):
    r = lax.broadcasted_iota(jnp.int32, (d, _H), 0)
    c = lax.broadcasted_iota(jnp.int32, (d, _H), 1)
    return (r // _DK == c).astype(_f32)


def _head_expand_mat(rows, d):
    r = lax.broadcasted_iota(jnp.int32, (rows, d), 0)
    c = lax.broadcasted_iota(jnp.int32, (rows, d), 1)
    return (c // _DK == r).astype(_f32)


def _dot(a, b):
    return jnp.dot(a, b, preferred_element_type=_f32)


def _pre_body(x0_ref, w0_ref, b0_ref, wq_ref, bq_ref, wk_ref, bk_ref,
              wv_ref, bv_ref, h_ref, tkv_ref, tq_ref):
    x = x0_ref[...]
    h = _dot(x, w0_ref[...]) + b0_ref[...]
    h_ref[...] = h
    k = _dot(h, wk_ref[...]) + bk_ref[...]
    v = _dot(h, wv_ref[...]) + bv_ref[...]
    tkv_ref[...] = jnp.concatenate([k, v], axis=1)
    tq_ref[...] = _dot(h, wq_ref[...]) + bq_ref[...]


def _pre_call(x0, w0, b0, qkvw):
    n, d = x0.shape[0], w0.shape[1]
    tn = 1000
    grid = (n // tn,)
    row = lambda i: (i, 0)
    full = lambda a: pl.BlockSpec(a.shape, lambda i: (0,) * a.ndim)
    return pl.pallas_call(
        _pre_body,
        grid=grid,
        in_specs=[pl.BlockSpec((tn, x0.shape[1]), row), full(w0), full(b0)]
        + [full(a) for a in qkvw],
        out_specs=[pl.BlockSpec((tn, d), row), pl.BlockSpec((tn, 2 * d), row),
                   pl.BlockSpec((tn, d), row)],
        out_shape=[jax.ShapeDtypeStruct((n, d), _f32),
                   jax.ShapeDtypeStruct((n, 2 * d), _f32),
                   jax.ShapeDtypeStruct((n, d), _f32)],
    )(x0, w0, b0, *qkvw)


def _edge_body(uniform_e, e_ref, gkv_ref, gq_ref, we_ref, be_ref, woe_ref,
               boe_ref, wf1_ref, bf1_ref, wf2_ref, bf2_ref, g1_ref, b1_ref,
               g2_ref, b2_ref, e3_ref, wvw_ref):
    d = gq_ref.shape[1]
    k = gkv_ref[:, :d]
    v = gkv_ref[:, d:]
    q = gq_ref[...]
    if uniform_e:
        e_row = e_ref[...]
        ep_row = _dot(e_row, we_ref[...]) + be_ref[...]
        e = jnp.broadcast_to(e_row, k.shape)
        ep = jnp.broadcast_to(ep_row, k.shape)
    else:
        e = e_ref[...]
        ep = _dot(e, we_ref[...]) + be_ref[...]
    score = k * q * ep * 0.25
    logits = jnp.clip(_dot(score, _head_sum_mat(d)), -5.0, 5.0)
    w = jnp.exp(logits)
    wfull = _dot(w, _head_expand_mat(_H, d))
    wvw_ref[...] = jnp.concatenate([wfull * v, wfull], axis=1)
    e_o = _dot(score, woe_ref[...]) + boe_ref[...]
    e2 = _ln(e + e_o, g1_ref[...], b1_ref[...])
    f = jnp.maximum(_dot(e2, wf1_ref[...]) + bf1_ref[...], 0.0)
    ef = _dot(f, wf2_ref[...]) + bf2_ref[...]
    e3_ref[...] = _ln(e2 + ef, g2_ref[...], b2_ref[...])


def _edge_call(e_arg, gkv, gq, ew, uniform_e):
    eedges, d = gq.shape
    t = 1280
    grid = (eedges // t,)
    row = lambda i: (i, 0)
    zero = lambda i: (0, 0)
    full = lambda a: pl.BlockSpec(a.shape, lambda i: (0,) * a.ndim)
    e_spec = (pl.BlockSpec((1, d), zero) if uniform_e
              else pl.BlockSpec((t, d), row))
    return pl.pallas_call(
        functools.partial(_edge_body, uniform_e),
        grid=grid,
        in_specs=[e_spec, pl.BlockSpec((t, 2 * d), row),
                  pl.BlockSpec((t, d), row)] + [full(a) for a in ew],
        out_specs=[pl.BlockSpec((t, d), row),
                   pl.BlockSpec((t, 2 * d), row)],
        out_shape=[jax.ShapeDtypeStruct((eedges, d), _f32),
                   jax.ShapeDtypeStruct((eedges, 2 * d), _f32)],
    )(e_arg, gkv, gq, *ew)


def _node_core(h, accn_ref, accd_ref, woh_ref, boh_ref, wf1_ref, bf1_ref,
               wf2_ref, bf2_ref, g1_ref, b1_ref, g2_ref, b2_ref):
    num = accn_ref[0] + accn_ref[1]
    denf = accd_ref[0] + accd_ref[1] + 1e-6
    h_att = num / denf
    h_o = _dot(h_att, woh_ref[...]) + boh_ref[...]
    h2 = _ln(h + h_o, g1_ref[...], b1_ref[...])
    f = jnp.maximum(_dot(h2, wf1_ref[...]) + bf1_ref[...], 0.0)
    hf = _dot(f, wf2_ref[...]) + bf2_ref[...]
    return _ln(h2 + hf, g2_ref[...], b2_ref[...])


def _node_body(h_ref, accn_ref, accd_ref, woh_ref, boh_ref, wf1_ref, bf1_ref,
               wf2_ref, bf2_ref, g1_ref, b1_ref, g2_ref, b2_ref, wq_ref,
               bq_ref, wk_ref, bk_ref, wv_ref, bv_ref, h3_ref, tkv_ref,
               tq_ref):
    h3 = _node_core(h_ref[...], accn_ref, accd_ref, woh_ref, boh_ref, wf1_ref,
                    bf1_ref, wf2_ref, bf2_ref, g1_ref, b1_ref, g2_ref, b2_ref)
    h3_ref[...] = h3
    k = _dot(h3, wk_ref[...]) + bk_ref[...]
    v = _dot(h3, wv_ref[...]) + bv_ref[...]
    tkv_ref[...] = jnp.concatenate([k, v], axis=1)
    tq_ref[...] = _dot(h3, wq_ref[...]) + bq_ref[...]


def _node_call(h, accn2, accd2, nw, qkvw):
    n, d = h.shape
    tn = 1000
    grid = (n // tn,)
    row = lambda i: (i, 0)
    full = lambda a: pl.BlockSpec(a.shape, lambda i: (0,) * a.ndim)
    return pl.pallas_call(
        _node_body,
        grid=grid,
        in_specs=[pl.BlockSpec((tn, d), row),
                  pl.BlockSpec((2, tn, d), lambda i: (0, i, 0)),
                  pl.BlockSpec((2, tn, d), lambda i: (0, i, 0))]
        + [full(a) for a in nw] + [full(a) for a in qkvw],
        out_specs=[pl.BlockSpec((tn, d), row), pl.BlockSpec((tn, 2 * d), row),
                   pl.BlockSpec((tn, d), row)],
        out_shape=[jax.ShapeDtypeStruct((n, d), _f32),
                   jax.ShapeDtypeStruct((n, 2 * d), _f32),
                   jax.ShapeDtypeStruct((n, d), _f32)],
    )(h, accn2, accd2, *nw, *qkvw)


def _node_final_body(h_ref, accn_ref, accd_ref, woh_ref, boh_ref, wf1_ref,
                     bf1_ref, wf2_ref, bf2_ref, g1_ref, b1_ref, g2_ref,
                     b2_ref, m1_ref, c1_ref, m2_ref, c2_ref, m3_ref, c3_ref,
                     nout_ref, hsum_ref):
    h3 = _node_core(h_ref[...], accn_ref, accd_ref, woh_ref, boh_ref, wf1_ref,
                    bf1_ref, wf2_ref, bf2_ref, g1_ref, b1_ref, g2_ref, b2_ref)
    x = jnp.maximum(_dot(h3, m1_ref[...]) + c1_ref[...], 0.0)
    x = jnp.maximum(_dot(x, m2_ref[...]) + c2_ref[...], 0.0)
    nout_ref[...] = _dot(x, m3_ref[...]) + c3_ref[...]

    @pl.when(pl.program_id(0) == 0)
    def _():
        hsum_ref[...] = jnp.zeros_like(hsum_ref)

    hsum_ref[...] += jnp.sum(h3, axis=0, keepdims=True)


def _node_final_call(h, accn2, accd2, nw, mlpw):
    n, d = h.shape
    tn = 1000
    grid = (n // tn,)
    row = lambda i: (i, 0)
    full = lambda a: pl.BlockSpec(a.shape, lambda i: (0,) * a.ndim)
    return pl.pallas_call(
        _node_final_body,
        grid=grid,
        in_specs=[pl.BlockSpec((tn, d), row),
                  pl.BlockSpec((2, tn, d), lambda i: (0, i, 0)),
                  pl.BlockSpec((2, tn, d), lambda i: (0, i, 0))]
        + [full(a) for a in nw] + [full(a) for a in mlpw],
        out_specs=[pl.BlockSpec((tn, 3), row),
                   pl.BlockSpec((1, d), lambda i: (0, 0))],
        out_shape=[jax.ShapeDtypeStruct((n, 3), _f32),
                   jax.ShapeDtypeStruct((1, d), _f32)],
    )(h, accn2, accd2, *nw, *mlpw)


def _graph_body(n_nodes, hsum_ref, m1_ref, c1_ref, m2_ref, c2_ref, m3_ref,
                c3_ref, gout_ref):
    hg = hsum_ref[...] * (1.0 / n_nodes)
    x = jnp.maximum(_dot(hg, m1_ref[...]) + c1_ref[...], 0.0)
    x = jnp.maximum(_dot(x, m2_ref[...]) + c2_ref[...], 0.0)
    gout_ref[...] = _dot(x, m3_ref[...]) + c3_ref[...]


def _graph_call(hsum, mlpw, n_nodes):
    return pl.pallas_call(
        functools.partial(_graph_body, n_nodes),
        out_shape=jax.ShapeDtypeStruct((1, 3), _f32),
    )(hsum, *mlpw)


def _sc_gather(tkv, tq, src, dst):
    e = src.shape[0]
    n, dkv = tkv.shape
    d = tq.shape[1]
    epw = e // _NW
    chunks = epw // _CHK
    mesh = plsc.VectorSubcoreMesh(core_axis_name="c", subcore_axis_name="s",
                                  num_cores=_NC, num_subcores=_NS)

    @functools.partial(
        pl.kernel,
        out_type=[jax.ShapeDtypeStruct((e, dkv), _f32),
                  jax.ShapeDtypeStruct((e, d), _f32)],
        mesh=mesh,
        scratch_types=[pltpu.VMEM((_CHK,), jnp.int32),
                       pltpu.VMEM((_CHK,), jnp.int32),
                       pltpu.VMEM((_CHK, dkv), _f32),
                       pltpu.VMEM((_CHK, d), _f32),
                       pltpu.SemaphoreType.DMA,
                       pltpu.SemaphoreType.DMA],
    )
    def k(tkv_hbm, tq_hbm, src_hbm, dst_hbm, gkv_hbm, gq_hbm, idx_s, idx_d,
          bkv, bq, sem1, sem2):
        wid = lax.axis_index("s") * _NC + lax.axis_index("c")
        base = wid * epw

        def body(j, carry):
            off = base + j * _CHK
            pltpu.sync_copy(src_hbm.at[pl.ds(off, _CHK)], idx_s)
            pltpu.sync_copy(dst_hbm.at[pl.ds(off, _CHK)], idx_d)
            cp1 = pltpu.async_copy(tkv_hbm.at[idx_s], bkv, sem1)
            cp2 = pltpu.async_copy(tq_hbm.at[idx_d], bq, sem2)
            cp1.wait()
            cp2.wait()
            pltpu.sync_copy(bkv, gkv_hbm.at[pl.ds(off, _CHK)])
            pltpu.sync_copy(bq, gq_hbm.at[pl.ds(off, _CHK)])
            return carry

        lax.fori_loop(0, chunks, body, 0)

    return k(tkv, tq, src, dst)


def _sc_scatter(wvw, coff, dst, n, zrow, seq):
    e = wvw.shape[0]
    d = 128
    epw = e // _NW
    chunks = epw // _CHK
    rows = n // _NS
    mesh = plsc.VectorSubcoreMesh(core_axis_name="c", subcore_axis_name="s",
                                  num_cores=_NC, num_subcores=_NS)

    @functools.partial(
        pl.kernel,
        out_type=jax.ShapeDtypeStruct((_NC * n, d), _f32),
        mesh=mesh,
        scratch_types=[pltpu.VMEM_SHARED((n, d), _f32),
                       pltpu.VMEM((_CHK,), jnp.int32),
                       pltpu.VMEM((_CHK,), jnp.int32),
                       pltpu.VMEM((_CHK, d), _f32),
                       pltpu.VMEM((2, _CHK, d), _f32),
                       pltpu.VMEM((2, _CHK), jnp.int32),
                       pltpu.SemaphoreType.DMA((2,)),
                       pltpu.SemaphoreType.DMA((2,)),
                       pltpu.SemaphoreType.DMA((2,))],
    )
    def k(wvw_hbm, dst_hbm, zrow_hbm, seq_hbm, acc_hbm, acc, idx, ridx, buf,
          dbuf, idx2, sem, isem, asem):
        cid = lax.axis_index("c")
        sid = lax.axis_index("s")
        zchunks = rows // _CHK

        pltpu.sync_copy(zrow_hbm, buf)

        def zbody(t, carry):
            roff = sid * rows + t * _CHK
            pltpu.sync_copy(seq_hbm.at[pl.ds(roff, _CHK)], ridx)
            pltpu.sync_copy(buf, acc.at[ridx])
            return carry

        lax.fori_loop(0, zchunks, zbody, 0)
        plsc.subcore_barrier()

        wid = sid * _NC + cid
        base = wid * epw

        def _load(j, slot):
            src = wvw_hbm.at[pl.ds(base + j * _CHK, _CHK), pl.ds(coff, d)]
            return pltpu.make_async_copy(src, dbuf.at[slot], sem.at[slot])

        def _iload(j, slot):
            src = dst_hbm.at[pl.ds(base + j * _CHK, _CHK)]
            return pltpu.make_async_copy(src, idx2.at[slot], isem.at[slot])

        def _add(slot):
            return pltpu.make_async_copy(dbuf.at[slot], acc.at[idx2.at[slot]],
                                         asem.at[slot])

        _load(0, 0).start()
        _iload(0, 0).start()

        def body(j, carry):
            m = j % 2
            nx = (j + 1) % 2

            @pl.when(j >= 1)
            def _():
                _add(nx).wait()

            @pl.when(j + 1 < chunks)
            def _():
                _load(j + 1, nx).start()
                _iload(j + 1, nx).start()

            _load(j, m).wait()
            _iload(j, m).wait()
            _add(m).start(add=True)
            return carry

        lax.fori_loop(0, chunks, body, 0)
        _add((chunks - 1) % 2).wait()
        plsc.subcore_barrier()

        def obody(t, carry):
            roff = sid * rows + t * _CHK
            pltpu.sync_copy(seq_hbm.at[pl.ds(roff, _CHK)], ridx)
            pltpu.sync_copy(acc.at[ridx], buf)
            pltpu.sync_copy(buf, acc_hbm.at[pl.ds(cid * n + roff, _CHK)])
            return carry

        lax.fori_loop(0, zchunks, obody, 0)

    return k(wvw, dst, zrow, seq)


def kernel(h, e, pos_enc, edge_index, params):
    n, d = h.shape[0], params['Wh'].shape[1]
    del e
    src = edge_index[0]
    dst = edge_index[1]
    lw = params['layers']
    n_layers = lw['Wq'].shape[0]
    r2 = lambda a: a.reshape(1, -1)

    pad = d // 8 - h.shape[1] - pos_enc.shape[1]
    x0 = jnp.concatenate([h, pos_enc, jnp.zeros((n, pad), _f32)], axis=1)
    w0 = jnp.concatenate(
        [params['Wh'], params['Wpe'], jnp.zeros((pad, d), _f32)], axis=0)
    b0 = r2(params['bh'] + params['bpe'])

    qkvw = lambda l: (lw['Wq'][l], r2(lw['bq'][l]), lw['Wk'][l],
                      r2(lw['bk'][l]), lw['Wv'][l], r2(lw['bv'][l]))
    hcur, tkv, tq = _pre_call(x0, w0, b0, qkvw(0))

    e_cur = params['Wee'][0:1] + r2(params['bee'])
    npad = ((n + _CHK * _NS - 1) // (_CHK * _NS)) * (_CHK * _NS)
    zrow = jnp.zeros((_CHK, d), _f32)
    seq = jnp.arange(npad, dtype=jnp.int32)

    nout = gout = None
    for l in range(n_layers):
        gkv, gq = _sc_gather(tkv, tq, src, dst)
        ew = (lw['We'][l], r2(lw['be'][l]), lw['Woe'][l], r2(lw['boe'][l]),
              lw['Wf1e'][l], r2(lw['bf1e'][l]), lw['Wf2e'][l],
              r2(lw['bf2e'][l]), r2(lw['ln1eg'][l]), r2(lw['ln1eb'][l]),
              r2(lw['ln2eg'][l]), r2(lw['ln2eb'][l]))
        e_cur, wvw = _edge_call(e_cur, gkv, gq, ew, uniform_e=(l == 0))
        accn2 = _sc_scatter(wvw, 0, dst, npad, zrow, seq).reshape(
            _NC, npad, -1)
        accd2 = _sc_scatter(wvw, d, dst, npad, zrow, seq).reshape(
            _NC, npad, -1)
        nw = (lw['Woh'][l], r2(lw['boh'][l]), lw['Wf1h'][l],
              r2(lw['bf1h'][l]), lw['Wf2h'][l], r2(lw['bf2h'][l]),
              r2(lw['ln1hg'][l]), r2(lw['ln1hb'][l]), r2(lw['ln2hg'][l]),
              r2(lw['ln2hb'][l]))
        if l < n_layers - 1:
            hcur, tkv, tq = _node_call(hcur, accn2, accd2, nw, qkvw(l + 1))
        else:
            mlpn = tuple(x for wb in params['mlp_n']
                         for x in (wb[0], r2(wb[1])))
            nout, hsum = _node_final_call(hcur, accn2, accd2, nw, mlpn)
            mlpg = tuple(x for wb in params['mlp_g']
                         for x in (wb[0], r2(wb[1])))
            gout = _graph_call(hsum, mlpg, n)

    return nout, gout.reshape(gout.shape[-1])


# speedup vs baseline: 36.9383x; 1.1728x over previous
import functools

import jax
import jax.numpy as jnp
from jax import lax
from jax.experimental import pallas as pl
from jax.experimental.pallas import tpu as pltpu
from jax.experimental.pallas import tpu_sc as plsc

_f32 = jnp.float32

_NC = 2
_NS = 16
_NW = _NC * _NS
_CHK = 80

_H = 8
_DK = 16


def _ln(x, g, b):
    m = jnp.mean(x, axis=-1, keepdims=True)
    xc = x - m
    v = jnp.mean(xc * xc, axis=-1, keepdims=True)
    return xc * lax.rsqrt(v + 1e-5) * g + b


def _head_sum_mat(d):
    r = lax.broadcasted_iota(jnp.int32, (d, _H), 0)
    c = lax.broadcasted_iota(jnp.int32, (d, _H), 1)
    return (r // _DK == c).astype(_f32)


def _head_expand_mat(rows, d):
    r = lax.broadcasted_iota(jnp.int32, (rows, d), 0)
    c = lax.broadcasted_iota(jnp.int32, (rows, d), 1)
    return (c // _DK == r).astype(_f32)


def _dot(a, b):
    return jnp.dot(a, b, preferred_element_type=_f32)


def _pre_body(x0_ref, w0_ref, b0_ref, wq_ref, bq_ref, wk_ref, bk_ref,
              wv_ref, bv_ref, h_ref, tkv_ref, tq_ref):
    x = x0_ref[...]
    h = _dot(x, w0_ref[...]) + b0_ref[...]
    h_ref[...] = h
    k = _dot(h, wk_ref[...]) + bk_ref[...]
    v = _dot(h, wv_ref[...]) + bv_ref[...]
    tkv_ref[...] = jnp.concatenate([k, v], axis=1)
    tq_ref[...] = _dot(h, wq_ref[...]) + bq_ref[...]


def _pre_call(x0, w0, b0, qkvw):
    n, d = x0.shape[0], w0.shape[1]
    tn = 1000
    grid = (n // tn,)
    row = lambda i: (i, 0)
    full = lambda a: pl.BlockSpec(a.shape, lambda i: (0,) * a.ndim)
    return pl.pallas_call(
        _pre_body,
        grid=grid,
        in_specs=[pl.BlockSpec((tn, x0.shape[1]), row), full(w0), full(b0)]
        + [full(a) for a in qkvw],
        out_specs=[pl.BlockSpec((tn, d), row), pl.BlockSpec((tn, 2 * d), row),
                   pl.BlockSpec((tn, d), row)],
        out_shape=[jax.ShapeDtypeStruct((n, d), _f32),
                   jax.ShapeDtypeStruct((n, 2 * d), _f32),
                   jax.ShapeDtypeStruct((n, d), _f32)],
    )(x0, w0, b0, *qkvw)


def _edge_body(uniform_e, e_ref, gkv_ref, gq_ref, we_ref, be_ref, woe_ref,
               boe_ref, wf1_ref, bf1_ref, wf2_ref, bf2_ref, g1_ref, b1_ref,
               g2_ref, b2_ref, e3_ref, wvw_ref):
    d = gq_ref.shape[1]
    k = gkv_ref[:, :d]
    v = gkv_ref[:, d:]
    q = gq_ref[...]
    if uniform_e:
        e_row = e_ref[...]
        ep_row = _dot(e_row, we_ref[...]) + be_ref[...]
        e = jnp.broadcast_to(e_row, k.shape)
        ep = jnp.broadcast_to(ep_row, k.shape)
    else:
        e = e_ref[...]
        ep = _dot(e, we_ref[...]) + be_ref[...]
    score = k * q * ep * 0.25
    logits = jnp.clip(_dot(score, _head_sum_mat(d)), -5.0, 5.0)
    w = jnp.exp(logits)
    wfull = _dot(w, _head_expand_mat(_H, d))
    wvw_ref[...] = jnp.concatenate([wfull * v, wfull], axis=1)
    e_o = _dot(score, woe_ref[...]) + boe_ref[...]
    e2 = _ln(e + e_o, g1_ref[...], b1_ref[...])
    f = jnp.maximum(_dot(e2, wf1_ref[...]) + bf1_ref[...], 0.0)
    ef = _dot(f, wf2_ref[...]) + bf2_ref[...]
    e3_ref[...] = _ln(e2 + ef, g2_ref[...], b2_ref[...])


def _edge_call(e_arg, gkv, gq, ew, uniform_e):
    eedges, d = gq.shape
    t = 1280
    grid = (eedges // t,)
    row = lambda i: (i, 0)
    zero = lambda i: (0, 0)
    full = lambda a: pl.BlockSpec(a.shape, lambda i: (0,) * a.ndim)
    e_spec = (pl.BlockSpec((1, d), zero) if uniform_e
              else pl.BlockSpec((t, d), row))
    return pl.pallas_call(
        functools.partial(_edge_body, uniform_e),
        grid=grid,
        in_specs=[e_spec, pl.BlockSpec((t, 2 * d), row),
                  pl.BlockSpec((t, d), row)] + [full(a) for a in ew],
        out_specs=[pl.BlockSpec((t, d), row),
                   pl.BlockSpec((t, 2 * d), row)],
        out_shape=[jax.ShapeDtypeStruct((eedges, d), _f32),
                   jax.ShapeDtypeStruct((eedges, 2 * d), _f32)],
    )(e_arg, gkv, gq, *ew)


def _node_core(h, accn_ref, accd_ref, woh_ref, boh_ref, wf1_ref, bf1_ref,
               wf2_ref, bf2_ref, g1_ref, b1_ref, g2_ref, b2_ref):
    num = accn_ref[0] + accn_ref[1]
    denf = accd_ref[0] + accd_ref[1] + 1e-6
    h_att = num / denf
    h_o = _dot(h_att, woh_ref[...]) + boh_ref[...]
    h2 = _ln(h + h_o, g1_ref[...], b1_ref[...])
    f = jnp.maximum(_dot(h2, wf1_ref[...]) + bf1_ref[...], 0.0)
    hf = _dot(f, wf2_ref[...]) + bf2_ref[...]
    return _ln(h2 + hf, g2_ref[...], b2_ref[...])


def _node_body(h_ref, accn_ref, accd_ref, woh_ref, boh_ref, wf1_ref, bf1_ref,
               wf2_ref, bf2_ref, g1_ref, b1_ref, g2_ref, b2_ref, wq_ref,
               bq_ref, wk_ref, bk_ref, wv_ref, bv_ref, h3_ref, tkv_ref,
               tq_ref):
    h3 = _node_core(h_ref[...], accn_ref, accd_ref, woh_ref, boh_ref, wf1_ref,
                    bf1_ref, wf2_ref, bf2_ref, g1_ref, b1_ref, g2_ref, b2_ref)
    h3_ref[...] = h3
    k = _dot(h3, wk_ref[...]) + bk_ref[...]
    v = _dot(h3, wv_ref[...]) + bv_ref[...]
    tkv_ref[...] = jnp.concatenate([k, v], axis=1)
    tq_ref[...] = _dot(h3, wq_ref[...]) + bq_ref[...]


def _node_call(h, accn2, accd2, nw, qkvw):
    n, d = h.shape
    tn = 1000
    grid = (n // tn,)
    row = lambda i: (i, 0)
    full = lambda a: pl.BlockSpec(a.shape, lambda i: (0,) * a.ndim)
    return pl.pallas_call(
        _node_body,
        grid=grid,
        in_specs=[pl.BlockSpec((tn, d), row),
                  pl.BlockSpec((2, tn, d), lambda i: (0, i, 0)),
                  pl.BlockSpec((2, tn, d), lambda i: (0, i, 0))]
        + [full(a) for a in nw] + [full(a) for a in qkvw],
        out_specs=[pl.BlockSpec((tn, d), row), pl.BlockSpec((tn, 2 * d), row),
                   pl.BlockSpec((tn, d), row)],
        out_shape=[jax.ShapeDtypeStruct((n, d), _f32),
                   jax.ShapeDtypeStruct((n, 2 * d), _f32),
                   jax.ShapeDtypeStruct((n, d), _f32)],
    )(h, accn2, accd2, *nw, *qkvw)


def _node_final_body(h_ref, accn_ref, accd_ref, woh_ref, boh_ref, wf1_ref,
                     bf1_ref, wf2_ref, bf2_ref, g1_ref, b1_ref, g2_ref,
                     b2_ref, m1_ref, c1_ref, m2_ref, c2_ref, m3_ref, c3_ref,
                     nout_ref, hsum_ref):
    h3 = _node_core(h_ref[...], accn_ref, accd_ref, woh_ref, boh_ref, wf1_ref,
                    bf1_ref, wf2_ref, bf2_ref, g1_ref, b1_ref, g2_ref, b2_ref)
    x = jnp.maximum(_dot(h3, m1_ref[...]) + c1_ref[...], 0.0)
    x = jnp.maximum(_dot(x, m2_ref[...]) + c2_ref[...], 0.0)
    nout_ref[...] = _dot(x, m3_ref[...]) + c3_ref[...]

    @pl.when(pl.program_id(0) == 0)
    def _():
        hsum_ref[...] = jnp.zeros_like(hsum_ref)

    hsum_ref[...] += jnp.sum(h3, axis=0, keepdims=True)


def _node_final_call(h, accn2, accd2, nw, mlpw):
    n, d = h.shape
    tn = 1000
    grid = (n // tn,)
    row = lambda i: (i, 0)
    full = lambda a: pl.BlockSpec(a.shape, lambda i: (0,) * a.ndim)
    return pl.pallas_call(
        _node_final_body,
        grid=grid,
        in_specs=[pl.BlockSpec((tn, d), row),
                  pl.BlockSpec((2, tn, d), lambda i: (0, i, 0)),
                  pl.BlockSpec((2, tn, d), lambda i: (0, i, 0))]
        + [full(a) for a in nw] + [full(a) for a in mlpw],
        out_specs=[pl.BlockSpec((tn, 3), row),
                   pl.BlockSpec((1, d), lambda i: (0, 0))],
        out_shape=[jax.ShapeDtypeStruct((n, 3), _f32),
                   jax.ShapeDtypeStruct((1, d), _f32)],
    )(h, accn2, accd2, *nw, *mlpw)


def _graph_body(n_nodes, hsum_ref, m1_ref, c1_ref, m2_ref, c2_ref, m3_ref,
                c3_ref, gout_ref):
    hg = hsum_ref[...] * (1.0 / n_nodes)
    x = jnp.maximum(_dot(hg, m1_ref[...]) + c1_ref[...], 0.0)
    x = jnp.maximum(_dot(x, m2_ref[...]) + c2_ref[...], 0.0)
    gout_ref[...] = _dot(x, m3_ref[...]) + c3_ref[...]


def _graph_call(hsum, mlpw, n_nodes):
    return pl.pallas_call(
        functools.partial(_graph_body, n_nodes),
        out_shape=jax.ShapeDtypeStruct((1, 3), _f32),
    )(hsum, *mlpw)


def _sc_gather(tkv, tq, src, dst):
    e = src.shape[0]
    n, dkv = tkv.shape
    d = tq.shape[1]
    epw = e // _NW
    chunks = epw // _CHK
    mesh = plsc.VectorSubcoreMesh(core_axis_name="c", subcore_axis_name="s",
                                  num_cores=_NC, num_subcores=_NS)

    @functools.partial(
        pl.kernel,
        out_type=[jax.ShapeDtypeStruct((e, dkv), _f32),
                  jax.ShapeDtypeStruct((e, d), _f32)],
        mesh=mesh,
        scratch_types=[pltpu.VMEM((2, _CHK), jnp.int32),
                       pltpu.VMEM((2, _CHK), jnp.int32),
                       pltpu.VMEM((2, _CHK, dkv), _f32),
                       pltpu.VMEM((2, _CHK, d), _f32),
                       pltpu.SemaphoreType.DMA((2,)),
                       pltpu.SemaphoreType.DMA((2,)),
                       pltpu.SemaphoreType.DMA((2,))],
    )
    def k(tkv_hbm, tq_hbm, src_hbm, dst_hbm, gkv_hbm, gq_hbm, idx_s, idx_d,
          bkv, bq, isem, gsem, wsem):
        wid = lax.axis_index("s") * _NC + lax.axis_index("c")
        base = wid * epw

        def _iload(j, slot):
            off = base + j * _CHK
            return (pltpu.make_async_copy(src_hbm.at[pl.ds(off, _CHK)],
                                          idx_s.at[slot], isem.at[slot]),
                    pltpu.make_async_copy(dst_hbm.at[pl.ds(off, _CHK)],
                                          idx_d.at[slot], isem.at[slot]))

        def _gather(slot):
            return (pltpu.make_async_copy(tkv_hbm.at[idx_s.at[slot]],
                                          bkv.at[slot], gsem.at[slot]),
                    pltpu.make_async_copy(tq_hbm.at[idx_d.at[slot]],
                                          bq.at[slot], gsem.at[slot]))

        def _write(j, slot):
            off = base + j * _CHK
            return (pltpu.make_async_copy(bkv.at[slot],
                                          gkv_hbm.at[pl.ds(off, _CHK)],
                                          wsem.at[slot]),
                    pltpu.make_async_copy(bq.at[slot],
                                          gq_hbm.at[pl.ds(off, _CHK)],
                                          wsem.at[slot]))

        def _start(pair):
            pair[0].start()
            pair[1].start()

        def _wait(pair):
            pair[0].wait()
            pair[1].wait()

        _start(_iload(0, 0))
        _wait(_iload(0, 0))
        _start(_gather(0))
        _start(_iload(1, 1))

        def body(j, carry):
            m = j % 2
            nx = (j + 1) % 2

            @pl.when(j >= 1)
            def _():
                _wait(_write(j - 1, nx))

            @pl.when(j + 1 < chunks)
            def _():
                _wait(_iload(j + 1, nx))
                _start(_gather(nx))

            _wait(_gather(m))

            @pl.when(j + 2 < chunks)
            def _():
                _start(_iload(j + 2, m))

            _start(_write(j, m))
            return carry

        lax.fori_loop(0, chunks, body, 0)
        _wait(_write(chunks - 1, (chunks - 1) % 2))

    return k(tkv, tq, src, dst)


def _sc_scatter(wvw, coff, dst, n, zrow, seq):
    e = wvw.shape[0]
    d = 128
    epw = e // _NW
    chunks = epw // _CHK
    rows = n // _NS
    mesh = plsc.VectorSubcoreMesh(core_axis_name="c", subcore_axis_name="s",
                                  num_cores=_NC, num_subcores=_NS)

    @functools.partial(
        pl.kernel,
        out_type=jax.ShapeDtypeStruct((_NC * n, d), _f32),
        mesh=mesh,
        scratch_types=[pltpu.VMEM_SHARED((n, d), _f32),
                       pltpu.VMEM((_CHK,), jnp.int32),
                       pltpu.VMEM((_CHK,), jnp.int32),
                       pltpu.VMEM((_CHK, d), _f32),
                       pltpu.VMEM((2, _CHK, d), _f32),
                       pltpu.VMEM((2, _CHK), jnp.int32),
                       pltpu.SemaphoreType.DMA((2,)),
                       pltpu.SemaphoreType.DMA((2,)),
                       pltpu.SemaphoreType.DMA((2,))],
    )
    def k(wvw_hbm, dst_hbm, zrow_hbm, seq_hbm, acc_hbm, acc, idx, ridx, buf,
          dbuf, idx2, sem, isem, asem):
        cid = lax.axis_index("c")
        sid = lax.axis_index("s")
        zchunks = rows // _CHK

        pltpu.sync_copy(zrow_hbm, buf)

        def zbody(t, carry):
            roff = sid * rows + t * _CHK
            pltpu.sync_copy(seq_hbm.at[pl.ds(roff, _CHK)], ridx)
            pltpu.sync_copy(buf, acc.at[ridx])
            return carry

        lax.fori_loop(0, zchunks, zbody, 0)
        plsc.subcore_barrier()

        wid = sid * _NC + cid
        base = wid * epw

        def _load(j, slot):
            src = wvw_hbm.at[pl.ds(base + j * _CHK, _CHK), pl.ds(coff, d)]
            return pltpu.make_async_copy(src, dbuf.at[slot], sem.at[slot])

        def _iload(j, slot):
            src = dst_hbm.at[pl.ds(base + j * _CHK, _CHK)]
            return pltpu.make_async_copy(src, idx2.at[slot], isem.at[slot])

        def _add(slot):
            return pltpu.make_async_copy(dbuf.at[slot], acc.at[idx2.at[slot]],
                                         asem.at[slot])

        _load(0, 0).start()
        _iload(0, 0).start()

        def body(j, carry):
            m = j % 2
            nx = (j + 1) % 2

            @pl.when(j >= 1)
            def _():
                _add(nx).wait()

            @pl.when(j + 1 < chunks)
            def _():
                _load(j + 1, nx).start()
                _iload(j + 1, nx).start()

            _load(j, m).wait()
            _iload(j, m).wait()
            _add(m).start(add=True)
            return carry

        lax.fori_loop(0, chunks, body, 0)
        _add((chunks - 1) % 2).wait()
        plsc.subcore_barrier()

        def obody(t, carry):
            roff = sid * rows + t * _CHK
            pltpu.sync_copy(seq_hbm.at[pl.ds(roff, _CHK)], ridx)
            pltpu.sync_copy(acc.at[ridx], buf)
            pltpu.sync_copy(buf, acc_hbm.at[pl.ds(cid * n + roff, _CHK)])
            return carry

        lax.fori_loop(0, zchunks, obody, 0)

    return k(wvw, dst, zrow, seq)


def kernel(h, e, pos_enc, edge_index, params):
    n, d = h.shape[0], params['Wh'].shape[1]
    del e
    src = edge_index[0]
    dst = edge_index[1]
    lw = params['layers']
    n_layers = lw['Wq'].shape[0]
    r2 = lambda a: a.reshape(1, -1)

    pad = d // 8 - h.shape[1] - pos_enc.shape[1]
    x0 = jnp.concatenate([h, pos_enc, jnp.zeros((n, pad), _f32)], axis=1)
    w0 = jnp.concatenate(
        [params['Wh'], params['Wpe'], jnp.zeros((pad, d), _f32)], axis=0)
    b0 = r2(params['bh'] + params['bpe'])

    qkvw = lambda l: (lw['Wq'][l], r2(lw['bq'][l]), lw['Wk'][l],
                      r2(lw['bk'][l]), lw['Wv'][l], r2(lw['bv'][l]))
    hcur, tkv, tq = _pre_call(x0, w0, b0, qkvw(0))

    e_cur = params['Wee'][0:1] + r2(params['bee'])
    npad = ((n + _CHK * _NS - 1) // (_CHK * _NS)) * (_CHK * _NS)
    zrow = jnp.zeros((_CHK, d), _f32)
    seq = jnp.arange(npad, dtype=jnp.int32)

    nout = gout = None
    for l in range(n_layers):
        gkv, gq = _sc_gather(tkv, tq, src, dst)
        ew = (lw['We'][l], r2(lw['be'][l]), lw['Woe'][l], r2(lw['boe'][l]),
              lw['Wf1e'][l], r2(lw['bf1e'][l]), lw['Wf2e'][l],
              r2(lw['bf2e'][l]), r2(lw['ln1eg'][l]), r2(lw['ln1eb'][l]),
              r2(lw['ln2eg'][l]), r2(lw['ln2eb'][l]))
        e_cur, wvw = _edge_call(e_cur, gkv, gq, ew, uniform_e=(l == 0))
        accn2 = _sc_scatter(wvw, 0, dst, npad, zrow, seq).reshape(
            _NC, npad, -1)
        accd2 = _sc_scatter(wvw, d, dst, npad, zrow, seq).reshape(
            _NC, npad, -1)
        nw = (lw['Woh'][l], r2(lw['boh'][l]), lw['Wf1h'][l],
              r2(lw['bf1h'][l]), lw['Wf2h'][l], r2(lw['bf2h'][l]),
              r2(lw['ln1hg'][l]), r2(lw['ln1hb'][l]), r2(lw['ln2hg'][l]),
              r2(lw['ln2hb'][l]))
        if l < n_layers - 1:
            hcur, tkv, tq = _node_call(hcur, accn2, accd2, nw, qkvw(l + 1))
        else:
            mlpn = tuple(x for wb in params['mlp_n']
                         for x in (wb[0], r2(wb[1])))
            nout, hsum = _node_final_call(hcur, accn2, accd2, nw, mlpn)
            mlpg = tuple(x for wb in params['mlp_g']
                         for x in (wb[0], r2(wb[1])))
            gout = _graph_call(hsum, mlpg, n)

    return nout, gout.reshape(gout.shape[-1])
